# baseline (speedup 1.0000x reference)
import functools

import jax
import jax.numpy as jnp
from jax import lax
from jax.experimental import pallas as pl
from jax.experimental.pallas import tpu as pltpu
from jax.experimental.pallas import tpu_sc as plsc

_NUM_EXPERT = 64
_D = 768
_CAP = 128
_SLOTS = _NUM_EXPERT * _CAP
_NW = 32
_NC = 2


def _gate_body(x_ref, gw_ref, gb_ref, g0_ref, e0_ref, e1_ref):
    x = x_ref[...]
    logits = jnp.dot(x, gw_ref[...], preferred_element_type=jnp.float32)
    logits = logits + gb_ref[0, 0, :][None, :]
    B = logits.shape[0]
    cols = lax.broadcasted_iota(jnp.int32, logits.shape, 1)
    v0 = jnp.max(logits, axis=1)
    e0 = jnp.min(jnp.where(logits == v0[:, None], cols, _NUM_EXPERT), axis=1)
    masked = jnp.where(cols == e0[:, None], -jnp.inf, logits)
    v1 = jnp.max(masked, axis=1)
    e1 = jnp.min(jnp.where((masked == v1[:, None]) & (cols != e0[:, None]),
                           cols, _NUM_EXPERT), axis=1)
    g0_ref[...] = 1.0 / (1.0 + jnp.exp(v1 - v0))
    e0_ref[...] = e0
    e1_ref[...] = e1


def _gate(x, gate_w, gate_b):
    N = x.shape[0]
    B = 1024
    gb3 = gate_b.reshape(1, 1, _NUM_EXPERT)
    return pl.pallas_call(
        _gate_body,
        grid=(N // B,),
        in_specs=[
            pl.BlockSpec((B, _D), lambda i: (i, 0)),
            pl.BlockSpec((_D, _NUM_EXPERT), lambda i: (0, 0)),
            pl.BlockSpec((1, 1, _NUM_EXPERT), lambda i: (0, 0, 0)),
        ],
        out_specs=[
            pl.BlockSpec((B,), lambda i: (i,)),
            pl.BlockSpec((B,), lambda i: (i,)),
            pl.BlockSpec((B,), lambda i: (i,)),
        ],
        out_shape=[
            jax.ShapeDtypeStruct((N,), jnp.float32),
            jax.ShapeDtypeStruct((N,), jnp.int32),
            jax.ShapeDtypeStruct((N,), jnp.int32),
        ],
    )(x, gate_w, gb3)


def _expert_body(disp_ref, w_ref, wgt_ref, b_ref, out_ref):
    i = pl.program_id(0)
    d = disp_ref[...]
    y = jnp.dot(d, w_ref[0], preferred_element_type=jnp.float32)
    y = y + b_ref[0, 0, :][None, :]
    delta = y - d
    wrow = wgt_ref[0, 0, :]
    r = lax.broadcasted_iota(jnp.int32, (_CAP, _CAP), 0)
    c = lax.broadcasted_iota(jnp.int32, (_CAP, _CAP), 1)
    diag = jnp.where(r == c, jnp.broadcast_to(wrow[None, :], (_CAP, _CAP)), 0.0)
    scaled = jnp.dot(diag, delta, preferred_element_type=jnp.float32)
    out_ref[...] = jnp.where(i == _NUM_EXPERT, 0.0, scaled)


def _expert(disp, expert_w, wgt_tbl, expert_b):
    wgt3 = wgt_tbl.reshape(_NUM_EXPERT, 1, _CAP)
    b3 = expert_b.reshape(_NUM_EXPERT, 1, _D)
    last = _NUM_EXPERT - 1
    return pl.pallas_call(
        _expert_body,
        grid=(_NUM_EXPERT + 1,),
        in_specs=[
            pl.BlockSpec((_CAP, _D), lambda i: (jnp.minimum(i, last), 0)),
            pl.BlockSpec((1, _D, _D), lambda i: (jnp.minimum(i, last), 0, 0)),
            pl.BlockSpec((1, 1, _CAP), lambda i: (jnp.minimum(i, last), 0, 0)),
            pl.BlockSpec((1, 1, _D), lambda i: (jnp.minimum(i, last), 0, 0)),
        ],
        out_specs=pl.BlockSpec((_CAP, _D), lambda i: (i, 0)),
        out_shape=jax.ShapeDtypeStruct(((_NUM_EXPERT + 1) * _CAP, _D),
                                       jnp.float32),
    )(disp, expert_w, wgt3, b3)


def _sc_gather(x, idx_tbl):
    N, D = x.shape
    per_w = _SLOTS // _NW
    CH = 64
    mesh = plsc.VectorSubcoreMesh(core_axis_name="c", subcore_axis_name="s")

    @functools.partial(
        pl.kernel, mesh=mesh,
        out_type=jax.ShapeDtypeStruct((_SLOTS, D), jnp.float32),
        scratch_types=[
            pltpu.VMEM((CH,), jnp.int32),
            pltpu.VMEM((CH, D), jnp.float32),
            pltpu.SemaphoreType.DMA,
        ],
    )
    def k(x_hbm, idx_hbm, out_hbm, idx_v, rows_v, sem):
        wid = lax.axis_index("s") * _NC + lax.axis_index("c")

        def body(c, carry):
            base = wid * per_w + c * CH
            pltpu.sync_copy(idx_hbm.at[pl.ds(base, CH)], idx_v)
            pltpu.async_copy(x_hbm.at[idx_v], rows_v, sem).wait()
            pltpu.sync_copy(rows_v, out_hbm.at[pl.ds(base, CH)])
            return carry

        lax.fori_loop(0, per_w // CH, body, 0)

    return k(x, idx_tbl)


def _sc_combine(x, delta, p0, p1):
    N, D = x.shape
    per_w = N // _NW
    CH = 32
    NV = D // 16
    mesh = plsc.VectorSubcoreMesh(core_axis_name="c", subcore_axis_name="s")

    @functools.partial(
        pl.kernel, mesh=mesh,
        out_type=jax.ShapeDtypeStruct((N, D), jnp.float32),
        scratch_types=[
            pltpu.VMEM((CH,), jnp.int32),
            pltpu.VMEM((CH,), jnp.int32),
            pltpu.VMEM((CH, D), jnp.float32),
            pltpu.VMEM((CH, D), jnp.float32),
            pltpu.VMEM((CH, D), jnp.float32),
            pltpu.SemaphoreType.DMA,
        ],
    )
    def k(x_hbm, delta_hbm, p0_hbm, p1_hbm, out_hbm,
          p0v, p1v, xv, d0v, d1v, sem):
        wid = lax.axis_index("s") * _NC + lax.axis_index("c")

        def chunk(c, carry):
            tok0 = wid * per_w + c * CH
            pltpu.sync_copy(x_hbm.at[pl.ds(tok0, CH)], xv)
            pltpu.sync_copy(p0_hbm.at[pl.ds(tok0, CH)], p0v)
            pltpu.sync_copy(p1_hbm.at[pl.ds(tok0, CH)], p1v)
            cp0 = pltpu.async_copy(delta_hbm.at[p0v], d0v, sem)
            cp1 = pltpu.async_copy(delta_hbm.at[p1v], d1v, sem)
            cp0.wait()
            cp1.wait()

            def row(r, carry2):
                for j in range(NV):
                    o = j * 16
                    xv[r, pl.ds(o, 16)] = (xv[r, pl.ds(o, 16)]
                                           + d0v[r, pl.ds(o, 16)]
                                           + d1v[r, pl.ds(o, 16)])
                return carry2

            lax.fori_loop(0, CH, row, 0)
            pltpu.sync_copy(xv, out_hbm.at[pl.ds(tok0, CH)])
            return carry

        lax.fori_loop(0, per_w // CH, chunk, 0)

    return k(x, delta, p0, p1)


@jax.jit
def kernel(moe_inp, gate_w, gate_b, expert_w, expert_b):
    x = moe_inp
    N = x.shape[0]
    n_slots = N * 2

    g0, e0, e1 = _gate(x, gate_w, gate_b)

    slot_expert = jnp.stack([e0, e1], axis=1).reshape(-1)
    slot_score = jnp.stack([g0, g0], axis=1).reshape(-1)
    sort_key = slot_expert.astype(jnp.float32) * 4.0 - slot_score
    order = jnp.argsort(sort_key)
    sorted_expert = slot_expert[order]
    counts = jnp.bincount(slot_expert, length=_NUM_EXPERT)
    starts = jnp.concatenate(
        [jnp.zeros((1,), counts.dtype), jnp.cumsum(counts)[:-1]])
    rank = jnp.arange(n_slots, dtype=jnp.int32) - starts[sorted_expert]
    keep_sorted = rank < _CAP
    loc_sorted = jnp.where(keep_sorted,
                           sorted_expert * _CAP + rank,
                           _SLOTS).astype(jnp.int32)

    pos = jnp.zeros((n_slots,), jnp.int32).at[order].set(loc_sorted)
    p0 = pos[0::2]
    p1 = pos[1::2]

    tok_sorted = (order // 2).astype(jnp.int32)
    g_sorted = jnp.where(order % 2 == 0, g0[tok_sorted], 1.0 - g0[tok_sorted])
    idx_tbl = jnp.zeros((_SLOTS,), jnp.int32).at[loc_sorted].set(
        tok_sorted, mode='drop')
    wgt_tbl = jnp.zeros((_SLOTS,), jnp.float32).at[loc_sorted].set(
        g_sorted, mode='drop')

    disp = _sc_gather(x, idx_tbl)
    delta = _expert(disp, expert_w, wgt_tbl, expert_b)
    return _sc_combine(x, delta, p0, p1)
